# baseline (speedup 1.0000x reference)
import jax
import jax.numpy as jnp
from jax import lax
from jax.experimental import pallas as pl
from jax.experimental.pallas import tpu as pltpu
from jax.experimental.pallas import tpu_sc as plsc

E = 23
K = 3
D = 768
H = 384
BT = 128
NT = 72
NW = 32
NEG = -1e30


def _gelu(v):
    return 0.5 * v * (1.0 + jax.lax.erf(v * 0.7071067811865476))


def _pack2(v):
    n = v.shape[1] // 2
    a = jax.lax.bitcast_convert_type(
        v[:, :n].astype(jnp.bfloat16).astype(jnp.float32), jnp.int32)
    b = jax.lax.bitcast_convert_type(
        v[:, n:].astype(jnp.bfloat16).astype(jnp.float32), jnp.int32)
    return jnp.bitwise_or(b, jax.lax.shift_right_logical(a, 16))


def _unpack2(u):
    a = jax.lax.bitcast_convert_type(jnp.left_shift(u, 16), jnp.float32)
    b = jax.lax.bitcast_convert_type(
        jnp.bitwise_and(u, jnp.int32(-65536)), jnp.float32)
    return jnp.concatenate([a, b], axis=1)


def _gate_kernel(x_ref, wg_ref, bg_ref, bias_ref,
                 topw_ref, pos_ref, te_ref, aux_ref, xpk_ref):
    T = x_ref.shape[0]
    x = x_ref[...]
    logits = jax.lax.dot_general(x, wg_ref[...], (((1,), (1,)), ((), ())),
                                 preferred_element_type=jnp.float32)
    gw = jax.nn.sigmoid(logits + bg_ref[...])
    lane = jax.lax.broadcasted_iota(jnp.int32, (T, E), 1)
    sel = gw + bias_ref[...]

    masks, vals = [], []
    for _ in range(K):
        m = jnp.max(sel, axis=1, keepdims=True)
        ismax = sel == m
        idx = jnp.min(jnp.where(ismax, lane, E), axis=1, keepdims=True)
        onek = lane == idx
        masks.append(onek)
        vals.append(jnp.sum(jnp.where(onek, gw, 0.0), axis=1, keepdims=True))
        sel = jnp.where(onek, NEG, sel)

    wsum = vals[0] + vals[1] + vals[2]
    o3 = (masks[0].astype(jnp.float32) + masks[1].astype(jnp.float32)
          + masks[2].astype(jnp.float32))
    counts_f = jnp.sum(o3, axis=0, keepdims=True)

    CB = 256
    tri = (jax.lax.broadcasted_iota(jnp.int32, (CB, CB), 0)
           > jax.lax.broadcasted_iota(jnp.int32, (CB, CB), 1)).astype(jnp.float32)
    carry = jnp.zeros((1, E), jnp.float32)
    rows = []
    for b in range(T // CB):
        ob = o3[b * CB:(b + 1) * CB]
        rows.append(jax.lax.dot_general(tri, ob, (((1,), (0,)), ((), ())),
                                        preferred_element_type=jnp.float32) + carry)
        carry = carry + jnp.sum(ob, axis=0, keepdims=True)
    cnt_before = jnp.concatenate(rows, axis=0)

    ntiles = jnp.floor((counts_f + (BT - 1)) / BT)
    triu = (jax.lax.broadcasted_iota(jnp.int32, (E, E), 0)
            < jax.lax.broadcasted_iota(jnp.int32, (E, E), 1)).astype(jnp.float32)
    poffset = BT * jax.lax.dot_general(ntiles, triu, (((1,), (0,)), ((), ())),
                                       preferred_element_type=jnp.float32)

    slot_f = cnt_before + poffset
    lane_k = jax.lax.broadcasted_iota(jnp.int32, (T, K), 1)
    topw = jnp.zeros((T, K), jnp.float32)
    pos = jnp.zeros((T, K), jnp.float32)
    for k in range(K):
        pos_k = jnp.sum(jnp.where(masks[k], slot_f, 0.0), axis=1, keepdims=True)
        topw = topw + jnp.where(lane_k == k, vals[k] / wsum, 0.0)
        pos = pos + jnp.where(lane_k == k, pos_k, 0.0)

    topw_ref[...] = topw
    pos_ref[...] = pos.astype(jnp.int32)

    excl = (poffset / BT).astype(jnp.int32)
    jj = jax.lax.broadcasted_iota(jnp.int32, (E, NT), 1)
    lemask = (jnp.broadcast_to(excl.reshape(E, 1), (E, NT)) <= jj)
    te_ref[...] = (jnp.sum(lemask.astype(jnp.float32), axis=0, keepdims=True)
                   - 1.0).astype(jnp.int32)

    gwn = gw / jnp.sum(gw, axis=1, keepdims=True)
    Pv = jnp.sum(gwn, axis=0, keepdims=True) / T
    Fv = E * counts_f / (K * T)
    aux_ref[...] = jnp.sum(Pv * Fv, keepdims=True)
    xpk_ref[...] = _pack2(x)


def _ffn_kernel(te_ref, xs_ref, w1_ref, b1_ref, w2_ref, b2_ref, ys_ref):
    x = _unpack2(xs_ref[...]).astype(jnp.bfloat16)
    w1 = w1_ref[...][0].astype(jnp.bfloat16)
    h = jax.lax.dot_general(x, w1, (((1,), (1,)), ((), ())),
                            preferred_element_type=jnp.float32) + b1_ref[...][0]
    h = _gelu(h).astype(jnp.bfloat16)
    w2 = w2_ref[...][0].astype(jnp.bfloat16)
    y = jax.lax.dot_general(h, w2, (((1,), (1,)), ((), ())),
                            preferred_element_type=jnp.float32) + b2_ref[...][0]
    ys_ref[...] = _pack2(y)


def _combine_kernel(x_ref, yg_ref, tw_ref, ws1_ref, bs1_ref, ws2_ref, bs2_ref,
                    o_ref):
    x = _unpack2(x_ref[...]).astype(jnp.bfloat16)
    ws1 = ws1_ref[...].astype(jnp.bfloat16)
    h = jax.lax.dot_general(x, ws1, (((1,), (1,)), ((), ())),
                            preferred_element_type=jnp.float32) + bs1_ref[...]
    h = _gelu(h).astype(jnp.bfloat16)
    ws2 = ws2_ref[...].astype(jnp.bfloat16)
    acc = jax.lax.dot_general(h, ws2, (((1,), (1,)), ((), ())),
                              preferred_element_type=jnp.float32) + bs2_ref[...]
    tw = tw_ref[...]
    yg = yg_ref[...]
    W2K = D // 2
    for k in range(K):
        acc = acc + tw[:, k:k + 1] * _unpack2(yg[:, k * W2K:(k + 1) * W2K])
    o_ref[...] = acc


def _disp_kernel(x_hbm, p0_hbm, p1_hbm, p2_hbm, xs_hbm,
                 rows_v, i0_v, i1_v, i2_v, sem):
    wid = lax.axis_index("s") * 2 + lax.axis_index("c")
    TW = rows_v.shape[0]
    base = wid * TW
    pltpu.sync_copy(x_hbm.at[pl.ds(base, TW)], rows_v)
    pltpu.sync_copy(p0_hbm.at[pl.ds(base, TW)], i0_v)
    pltpu.sync_copy(p1_hbm.at[pl.ds(base, TW)], i1_v)
    pltpu.sync_copy(p2_hbm.at[pl.ds(base, TW)], i2_v)
    d0 = pltpu.async_copy(rows_v, xs_hbm.at[i0_v], sem)
    d1 = pltpu.async_copy(rows_v, xs_hbm.at[i1_v], sem)
    d2 = pltpu.async_copy(rows_v, xs_hbm.at[i2_v], sem)
    d0.wait()
    d1.wait()
    d2.wait()


def _gath_kernel(ys_hbm, pf_hbm, yg_hbm, r0_v, r1_v, i0_v, i1_v, sem):
    wid = lax.axis_index("s") * 2 + lax.axis_index("c")
    CW = r0_v.shape[0]
    nchunk = pf_hbm.shape[0] // (NW * CW)
    rows = (r0_v, r1_v)
    idxs = (i0_v, i1_v)
    base0 = wid * (CW * nchunk)
    pltpu.sync_copy(pf_hbm.at[pl.ds(base0, CW)], i0_v)
    pend = pltpu.async_copy(ys_hbm.at[i0_v], r0_v, sem)
    for c in range(nchunk):
        if c + 1 < nchunk:
            nb = base0 + (c + 1) * CW
            pltpu.sync_copy(pf_hbm.at[pl.ds(nb, CW)], idxs[(c + 1) % 2])
            nxt = pltpu.async_copy(ys_hbm.at[idxs[(c + 1) % 2]],
                                   rows[(c + 1) % 2], sem)
        pend.wait()
        pltpu.sync_copy(rows[c % 2], yg_hbm.at[pl.ds(base0 + c * CW, CW)])
        if c + 1 < nchunk:
            pend = nxt


def kernel(x, Wg, bg, W1, b1, W2, b2, Ws1, bs1, Ws2, bs2, bias):
    o_shape = x.shape
    x2 = x.reshape(-1, D)
    T = x2.shape[0]
    PADN = NT * BT

    topw, pos, te, aux, xpk = pl.pallas_call(
        _gate_kernel,
        out_shape=[
            jax.ShapeDtypeStruct((T, K), jnp.float32),
            jax.ShapeDtypeStruct((T, K), jnp.int32),
            jax.ShapeDtypeStruct((1, NT), jnp.int32),
            jax.ShapeDtypeStruct((1, 1), jnp.float32),
            jax.ShapeDtypeStruct((T, D // 2), jnp.int32),
        ],
    )(x2, Wg, bg.reshape(1, E), bias.reshape(1, E))
    tile_expert = te.reshape(NT)

    TW = T // NW
    mesh = plsc.VectorSubcoreMesh(core_axis_name="c", subcore_axis_name="s")
    xs = pl.kernel(
        _disp_kernel,
        out_type=jax.ShapeDtypeStruct((PADN, D // 2), jnp.int32),
        mesh=mesh,
        scratch_types=[
            pltpu.VMEM((TW, D // 2), jnp.int32),
            pltpu.VMEM((TW,), jnp.int32),
            pltpu.VMEM((TW,), jnp.int32),
            pltpu.VMEM((TW,), jnp.int32),
            pltpu.SemaphoreType.DMA,
        ],
    )(xpk, pos[:, 0], pos[:, 1], pos[:, 2])

    grid_spec = pltpu.PrefetchScalarGridSpec(
        num_scalar_prefetch=1,
        grid=(NT,),
        in_specs=[
            pl.BlockSpec((BT, D // 2), lambda j, te: (j, 0)),
            pl.BlockSpec((1, H, D), lambda j, te: (te[j], 0, 0)),
            pl.BlockSpec((1, 1, H), lambda j, te: (te[j], 0, 0)),
            pl.BlockSpec((1, D, H), lambda j, te: (te[j], 0, 0)),
            pl.BlockSpec((1, 1, D), lambda j, te: (te[j], 0, 0)),
        ],
        out_specs=pl.BlockSpec((BT, D // 2), lambda j, te: (j, 0)),
    )
    ys = pl.pallas_call(
        _ffn_kernel,
        grid_spec=grid_spec,
        out_shape=jax.ShapeDtypeStruct((PADN, D // 2), jnp.int32),
    )(tile_expert, xs, W1, b1.reshape(E, 1, H), W2, b2.reshape(E, 1, D))

    CW = 64
    yg = pl.kernel(
        _gath_kernel,
        out_type=jax.ShapeDtypeStruct((T * K, D // 2), jnp.int32),
        mesh=mesh,
        scratch_types=[
            pltpu.VMEM((CW, D // 2), jnp.int32),
            pltpu.VMEM((CW, D // 2), jnp.int32),
            pltpu.VMEM((CW,), jnp.int32),
            pltpu.VMEM((CW,), jnp.int32),
            pltpu.SemaphoreType.DMA,
        ],
    )(ys, pos.reshape(T * K))

    BTC = 256
    out = pl.pallas_call(
        _combine_kernel,
        grid=(T // BTC,),
        in_specs=[
            pl.BlockSpec((BTC, D // 2), lambda i: (i, 0)),
            pl.BlockSpec((BTC, K * D // 2), lambda i: (i, 0)),
            pl.BlockSpec((BTC, K), lambda i: (i, 0)),
            pl.BlockSpec((H, D), lambda i: (0, 0)),
            pl.BlockSpec((1, H), lambda i: (0, 0)),
            pl.BlockSpec((D, H), lambda i: (0, 0)),
            pl.BlockSpec((1, D), lambda i: (0, 0)),
        ],
        out_specs=pl.BlockSpec((BTC, D), lambda i: (i, 0)),
        out_shape=jax.ShapeDtypeStruct((T, D), jnp.float32),
    )(xpk, yg.reshape(T, K * D // 2), topw, Ws1, bs1.reshape(1, H),
      Ws2, bs2.reshape(1, D))

    return out.reshape(o_shape), aux[0, 0]


# speedup vs baseline: 1.0156x; 1.0156x over previous
import jax
import jax.numpy as jnp
from jax import lax
from jax.experimental import pallas as pl
from jax.experimental.pallas import tpu as pltpu
from jax.experimental.pallas import tpu_sc as plsc

E = 23
K = 3
D = 768
H = 384
BT = 128
NT = 72
NW = 32
NEG = -1e30


def _gelu(v):
    return 0.5 * v * (1.0 + jax.lax.erf(v * 0.7071067811865476))


def _pack2(v):
    n = v.shape[1] // 2
    a = jax.lax.bitcast_convert_type(
        v[:, :n].astype(jnp.bfloat16).astype(jnp.float32), jnp.int32)
    b = jax.lax.bitcast_convert_type(
        v[:, n:].astype(jnp.bfloat16).astype(jnp.float32), jnp.int32)
    return jnp.bitwise_or(b, jax.lax.shift_right_logical(a, 16))


def _unpack2(u):
    a = jax.lax.bitcast_convert_type(jnp.left_shift(u, 16), jnp.float32)
    b = jax.lax.bitcast_convert_type(
        jnp.bitwise_and(u, jnp.int32(-65536)), jnp.float32)
    return jnp.concatenate([a, b], axis=1)


def _gate_kernel(x_ref, wg_ref, bg_ref, bias_ref,
                 topw_ref, pos_ref, te_ref, aux_ref, xpk_ref):
    T = x_ref.shape[0]
    x = x_ref[...]
    logits = jax.lax.dot_general(x, wg_ref[...], (((1,), (1,)), ((), ())),
                                 preferred_element_type=jnp.float32)
    gw = jax.nn.sigmoid(logits + bg_ref[...])
    lane = jax.lax.broadcasted_iota(jnp.int32, (T, E), 1)
    sel = gw + bias_ref[...]

    masks, vals = [], []
    for _ in range(K):
        m = jnp.max(sel, axis=1, keepdims=True)
        ismax = sel == m
        idx = jnp.min(jnp.where(ismax, lane, E), axis=1, keepdims=True)
        onek = lane == idx
        masks.append(onek)
        vals.append(jnp.sum(jnp.where(onek, gw, 0.0), axis=1, keepdims=True))
        sel = jnp.where(onek, NEG, sel)

    wsum = vals[0] + vals[1] + vals[2]
    o3 = (masks[0].astype(jnp.float32) + masks[1].astype(jnp.float32)
          + masks[2].astype(jnp.float32))
    counts_f = jnp.sum(o3, axis=0, keepdims=True)

    CB = 256
    tri = (jax.lax.broadcasted_iota(jnp.int32, (CB, CB), 0)
           > jax.lax.broadcasted_iota(jnp.int32, (CB, CB), 1)).astype(jnp.float32)
    carry = jnp.zeros((1, E), jnp.float32)
    rows = []
    for b in range(T // CB):
        ob = o3[b * CB:(b + 1) * CB]
        rows.append(jax.lax.dot_general(tri, ob, (((1,), (0,)), ((), ())),
                                        preferred_element_type=jnp.float32) + carry)
        carry = carry + jnp.sum(ob, axis=0, keepdims=True)
    cnt_before = jnp.concatenate(rows, axis=0)

    ntiles = jnp.floor((counts_f + (BT - 1)) / BT)
    triu = (jax.lax.broadcasted_iota(jnp.int32, (E, E), 0)
            < jax.lax.broadcasted_iota(jnp.int32, (E, E), 1)).astype(jnp.float32)
    poffset = BT * jax.lax.dot_general(ntiles, triu, (((1,), (0,)), ((), ())),
                                       preferred_element_type=jnp.float32)

    slot_f = cnt_before + poffset
    lane_k = jax.lax.broadcasted_iota(jnp.int32, (T, K), 1)
    topw = jnp.zeros((T, K), jnp.float32)
    pos = jnp.zeros((T, K), jnp.float32)
    for k in range(K):
        pos_k = jnp.sum(jnp.where(masks[k], slot_f, 0.0), axis=1, keepdims=True)
        topw = topw + jnp.where(lane_k == k, vals[k] / wsum, 0.0)
        pos = pos + jnp.where(lane_k == k, pos_k, 0.0)

    topw_ref[...] = topw
    pos_ref[...] = pos.astype(jnp.int32)

    excl = (poffset / BT).astype(jnp.int32)
    jj = jax.lax.broadcasted_iota(jnp.int32, (E, NT), 1)
    lemask = (jnp.broadcast_to(excl.reshape(E, 1), (E, NT)) <= jj)
    te_ref[...] = (jnp.sum(lemask.astype(jnp.float32), axis=0, keepdims=True)
                   - 1.0).astype(jnp.int32)

    gwn = gw / jnp.sum(gw, axis=1, keepdims=True)
    Pv = jnp.sum(gwn, axis=0, keepdims=True) / T
    Fv = E * counts_f / (K * T)
    aux_ref[...] = jnp.sum(Pv * Fv, keepdims=True)
    xpk_ref[...] = _pack2(x)


def _ffn_kernel(te_ref, xs_ref, w1_ref, b1_ref, w2_ref, b2_ref, ys_ref):
    x = _unpack2(xs_ref[...]).astype(jnp.bfloat16)
    w1 = w1_ref[...][0].astype(jnp.bfloat16)
    h = jax.lax.dot_general(x, w1, (((1,), (1,)), ((), ())),
                            preferred_element_type=jnp.float32) + b1_ref[...][0]
    h = _gelu(h).astype(jnp.bfloat16)
    w2 = w2_ref[...][0].astype(jnp.bfloat16)
    y = jax.lax.dot_general(h, w2, (((1,), (1,)), ((), ())),
                            preferred_element_type=jnp.float32) + b2_ref[...][0]
    ys_ref[...] = _pack2(y)


def _combine_kernel(x_ref, yg_ref, tw_ref, ws1_ref, bs1_ref, ws2_ref, bs2_ref,
                    o_ref):
    x = _unpack2(x_ref[...]).astype(jnp.bfloat16)
    ws1 = ws1_ref[...].astype(jnp.bfloat16)
    h = jax.lax.dot_general(x, ws1, (((1,), (1,)), ((), ())),
                            preferred_element_type=jnp.float32) + bs1_ref[...]
    h = _gelu(h).astype(jnp.bfloat16)
    ws2 = ws2_ref[...].astype(jnp.bfloat16)
    acc = jax.lax.dot_general(h, ws2, (((1,), (1,)), ((), ())),
                              preferred_element_type=jnp.float32) + bs2_ref[...]
    tw = tw_ref[...]
    yg = yg_ref[...]
    W2K = D // 2
    for k in range(K):
        acc = acc + tw[:, k:k + 1] * _unpack2(yg[:, k * W2K:(k + 1) * W2K])
    o_ref[...] = acc


def _disp_kernel(x_hbm, p0_hbm, p1_hbm, p2_hbm, xs_hbm,
                 rows_v, i0_v, i1_v, i2_v, sem):
    wid = lax.axis_index("s") * 2 + lax.axis_index("c")
    TW = rows_v.shape[0]
    base = wid * TW
    pltpu.sync_copy(x_hbm.at[pl.ds(base, TW)], rows_v)
    pltpu.sync_copy(p0_hbm.at[pl.ds(base, TW)], i0_v)
    pltpu.sync_copy(p1_hbm.at[pl.ds(base, TW)], i1_v)
    pltpu.sync_copy(p2_hbm.at[pl.ds(base, TW)], i2_v)
    d0 = pltpu.async_copy(rows_v, xs_hbm.at[i0_v], sem)
    d1 = pltpu.async_copy(rows_v, xs_hbm.at[i1_v], sem)
    d2 = pltpu.async_copy(rows_v, xs_hbm.at[i2_v], sem)
    d0.wait()
    d1.wait()
    d2.wait()


def _gath_kernel(ys_hbm, pf_hbm, yg_hbm, r0_v, r1_v, i0_v, i1_v, sem):
    wid = lax.axis_index("s") * 2 + lax.axis_index("c")
    CW = r0_v.shape[0]
    nchunk = pf_hbm.shape[0] // (NW * CW)
    rows = (r0_v, r1_v)
    idxs = (i0_v, i1_v)
    base0 = wid * (CW * nchunk)
    pltpu.sync_copy(pf_hbm.at[pl.ds(base0, CW)], i0_v)
    pend = pltpu.async_copy(ys_hbm.at[i0_v], r0_v, sem)
    for c in range(nchunk):
        if c + 1 < nchunk:
            nb = base0 + (c + 1) * CW
            pltpu.sync_copy(pf_hbm.at[pl.ds(nb, CW)], idxs[(c + 1) % 2])
            nxt = pltpu.async_copy(ys_hbm.at[idxs[(c + 1) % 2]],
                                   rows[(c + 1) % 2], sem)
        pend.wait()
        pltpu.sync_copy(rows[c % 2], yg_hbm.at[pl.ds(base0 + c * CW, CW)])
        if c + 1 < nchunk:
            pend = nxt


def kernel(x, Wg, bg, W1, b1, W2, b2, Ws1, bs1, Ws2, bs2, bias):
    o_shape = x.shape
    x2 = x.reshape(-1, D)
    T = x2.shape[0]
    PADN = NT * BT

    topw, pos, te, aux, xpk = pl.pallas_call(
        _gate_kernel,
        out_shape=[
            jax.ShapeDtypeStruct((T, K), jnp.float32),
            jax.ShapeDtypeStruct((T, K), jnp.int32),
            jax.ShapeDtypeStruct((1, NT), jnp.int32),
            jax.ShapeDtypeStruct((1, 1), jnp.float32),
            jax.ShapeDtypeStruct((T, D // 2), jnp.int32),
        ],
    )(x2, Wg, bg.reshape(1, E), bias.reshape(1, E))
    tile_expert = te.reshape(NT)

    TW = T // NW
    mesh = plsc.VectorSubcoreMesh(core_axis_name="c", subcore_axis_name="s")
    xs = pl.kernel(
        _disp_kernel,
        out_type=jax.ShapeDtypeStruct((PADN, D // 2), jnp.int32),
        mesh=mesh,
        scratch_types=[
            pltpu.VMEM((TW, D // 2), jnp.int32),
            pltpu.VMEM((TW,), jnp.int32),
            pltpu.VMEM((TW,), jnp.int32),
            pltpu.VMEM((TW,), jnp.int32),
            pltpu.SemaphoreType.DMA,
        ],
    )(xpk, pos[:, 0], pos[:, 1], pos[:, 2])

    grid_spec = pltpu.PrefetchScalarGridSpec(
        num_scalar_prefetch=1,
        grid=(NT,),
        in_specs=[
            pl.BlockSpec((BT, D // 2), lambda j, te: (j, 0)),
            pl.BlockSpec((1, H, D), lambda j, te: (te[j], 0, 0)),
            pl.BlockSpec((1, 1, H), lambda j, te: (te[j], 0, 0)),
            pl.BlockSpec((1, D, H), lambda j, te: (te[j], 0, 0)),
            pl.BlockSpec((1, 1, D), lambda j, te: (te[j], 0, 0)),
        ],
        out_specs=pl.BlockSpec((BT, D // 2), lambda j, te: (j, 0)),
    )
    ys = pl.pallas_call(
        _ffn_kernel,
        grid_spec=grid_spec,
        out_shape=jax.ShapeDtypeStruct((PADN, D // 2), jnp.int32),
    )(tile_expert, xs, W1, b1.reshape(E, 1, H), W2, b2.reshape(E, 1, D))

    CW = 64
    yg = pl.kernel(
        _gath_kernel,
        out_type=jax.ShapeDtypeStruct((T * K, D // 2), jnp.int32),
        mesh=mesh,
        scratch_types=[
            pltpu.VMEM((CW, D // 2), jnp.int32),
            pltpu.VMEM((CW, D // 2), jnp.int32),
            pltpu.VMEM((CW,), jnp.int32),
            pltpu.VMEM((CW,), jnp.int32),
            pltpu.SemaphoreType.DMA,
        ],
    )(ys, pos.reshape(T * K))

    BTC = 512
    out = pl.pallas_call(
        _combine_kernel,
        grid=(T // BTC,),
        in_specs=[
            pl.BlockSpec((BTC, D // 2), lambda i: (i, 0)),
            pl.BlockSpec((BTC, K * D // 2), lambda i: (i, 0)),
            pl.BlockSpec((BTC, K), lambda i: (i, 0)),
            pl.BlockSpec((H, D), lambda i: (0, 0)),
            pl.BlockSpec((1, H), lambda i: (0, 0)),
            pl.BlockSpec((D, H), lambda i: (0, 0)),
            pl.BlockSpec((1, D), lambda i: (0, 0)),
        ],
        out_specs=pl.BlockSpec((BTC, D), lambda i: (i, 0)),
        out_shape=jax.ShapeDtypeStruct((T, D), jnp.float32),
    )(xpk, yg.reshape(T, K * D // 2), topw, Ws1, bs1.reshape(1, H),
      Ws2, bs2.reshape(1, D))

    return out.reshape(o_shape), aux[0, 0]


# speedup vs baseline: 1.0209x; 1.0052x over previous
import jax
import jax.numpy as jnp
from jax import lax
from jax.experimental import pallas as pl
from jax.experimental.pallas import tpu as pltpu
from jax.experimental.pallas import tpu_sc as plsc

E = 23
K = 3
D = 768
H = 384
BT = 128
NT = 72
NW = 32
NEG = -1e30


def _gelu(v):
    return 0.5 * v * (1.0 + jax.lax.erf(v * 0.7071067811865476))


def _pack2(v):
    n = v.shape[1] // 2
    a = jax.lax.bitcast_convert_type(
        v[:, :n].astype(jnp.bfloat16).astype(jnp.float32), jnp.int32)
    b = jax.lax.bitcast_convert_type(
        v[:, n:].astype(jnp.bfloat16).astype(jnp.float32), jnp.int32)
    return jnp.bitwise_or(b, jax.lax.shift_right_logical(a, 16))


def _unpack2(u):
    a = jax.lax.bitcast_convert_type(jnp.left_shift(u, 16), jnp.float32)
    b = jax.lax.bitcast_convert_type(
        jnp.bitwise_and(u, jnp.int32(-65536)), jnp.float32)
    return jnp.concatenate([a, b], axis=1)


def _gate_kernel(x_ref, wg_ref, bg_ref, bias_ref,
                 topw_ref, pos_ref, te_ref, aux_ref, xpk_ref):
    T = x_ref.shape[0]
    x = x_ref[...]
    logits = jax.lax.dot_general(x, wg_ref[...], (((1,), (1,)), ((), ())),
                                 preferred_element_type=jnp.float32)
    gw = jax.nn.sigmoid(logits + bg_ref[...])
    lane = jax.lax.broadcasted_iota(jnp.int32, (T, E), 1)
    sel = gw + bias_ref[...]

    masks, vals = [], []
    for _ in range(K):
        m = jnp.max(sel, axis=1, keepdims=True)
        ismax = sel == m
        idx = jnp.min(jnp.where(ismax, lane, E), axis=1, keepdims=True)
        onek = lane == idx
        masks.append(onek)
        vals.append(jnp.sum(jnp.where(onek, gw, 0.0), axis=1, keepdims=True))
        sel = jnp.where(onek, NEG, sel)

    wsum = vals[0] + vals[1] + vals[2]
    o3 = (masks[0].astype(jnp.float32) + masks[1].astype(jnp.float32)
          + masks[2].astype(jnp.float32))
    counts_f = jnp.sum(o3, axis=0, keepdims=True)

    CB = 256
    tri = (jax.lax.broadcasted_iota(jnp.int32, (CB, CB), 0)
           > jax.lax.broadcasted_iota(jnp.int32, (CB, CB), 1)).astype(jnp.float32)
    carry = jnp.zeros((1, E), jnp.float32)
    rows = []
    for b in range(T // CB):
        ob = o3[b * CB:(b + 1) * CB]
        rows.append(jax.lax.dot_general(tri, ob, (((1,), (0,)), ((), ())),
                                        preferred_element_type=jnp.float32) + carry)
        carry = carry + jnp.sum(ob, axis=0, keepdims=True)
    cnt_before = jnp.concatenate(rows, axis=0)

    ntiles = jnp.floor((counts_f + (BT - 1)) / BT)
    triu = (jax.lax.broadcasted_iota(jnp.int32, (E, E), 0)
            < jax.lax.broadcasted_iota(jnp.int32, (E, E), 1)).astype(jnp.float32)
    poffset = BT * jax.lax.dot_general(ntiles, triu, (((1,), (0,)), ((), ())),
                                       preferred_element_type=jnp.float32)

    slot_f = cnt_before + poffset
    lane_k = jax.lax.broadcasted_iota(jnp.int32, (T, K), 1)
    topw = jnp.zeros((T, K), jnp.float32)
    pos = jnp.zeros((T, K), jnp.float32)
    for k in range(K):
        pos_k = jnp.sum(jnp.where(masks[k], slot_f, 0.0), axis=1, keepdims=True)
        topw = topw + jnp.where(lane_k == k, vals[k] / wsum, 0.0)
        pos = pos + jnp.where(lane_k == k, pos_k, 0.0)

    topw_ref[...] = topw
    pos_ref[...] = pos.astype(jnp.int32)

    excl = (poffset / BT).astype(jnp.int32)
    jj = jax.lax.broadcasted_iota(jnp.int32, (E, NT), 1)
    lemask = (jnp.broadcast_to(excl.reshape(E, 1), (E, NT)) <= jj)
    te_ref[...] = (jnp.sum(lemask.astype(jnp.float32), axis=0, keepdims=True)
                   - 1.0).astype(jnp.int32)

    gwn = gw / jnp.sum(gw, axis=1, keepdims=True)
    Pv = jnp.sum(gwn, axis=0, keepdims=True) / T
    Fv = E * counts_f / (K * T)
    aux_ref[...] = jnp.sum(Pv * Fv, keepdims=True)
    xpk_ref[...] = _pack2(x)


def _ffn_kernel(te_ref, xs_ref, w1_ref, b1_ref, w2_ref, b2_ref, ys_ref):
    x = _unpack2(xs_ref[...]).astype(jnp.bfloat16)
    w1 = w1_ref[...][0].astype(jnp.bfloat16)
    h = jax.lax.dot_general(x, w1, (((1,), (1,)), ((), ())),
                            preferred_element_type=jnp.float32) + b1_ref[...][0]
    h = _gelu(h).astype(jnp.bfloat16)
    w2 = w2_ref[...][0].astype(jnp.bfloat16)
    y = jax.lax.dot_general(h, w2, (((1,), (1,)), ((), ())),
                            preferred_element_type=jnp.float32) + b2_ref[...][0]
    ys_ref[...] = _pack2(y)


def _combine_kernel(x_ref, yg_ref, tw_ref, ws1_ref, bs1_ref, ws2_ref, bs2_ref,
                    o_ref):
    x = _unpack2(x_ref[...]).astype(jnp.bfloat16)
    ws1 = ws1_ref[...].astype(jnp.bfloat16)
    h = jax.lax.dot_general(x, ws1, (((1,), (1,)), ((), ())),
                            preferred_element_type=jnp.float32) + bs1_ref[...]
    h = _gelu(h).astype(jnp.bfloat16)
    ws2 = ws2_ref[...].astype(jnp.bfloat16)
    acc = jax.lax.dot_general(h, ws2, (((1,), (1,)), ((), ())),
                              preferred_element_type=jnp.float32) + bs2_ref[...]
    tw = tw_ref[...]
    yg = yg_ref[...]
    W2K = D // 2
    for k in range(K):
        acc = acc + tw[:, k:k + 1] * _unpack2(yg[:, k * W2K:(k + 1) * W2K])
    o_ref[...] = acc


def _disp_kernel(x_hbm, p0_hbm, p1_hbm, p2_hbm, xs_hbm,
                 rows_v, i0_v, i1_v, i2_v, sem):
    wid = lax.axis_index("s") * 2 + lax.axis_index("c")
    TW = rows_v.shape[0]
    base = wid * TW
    pltpu.sync_copy(x_hbm.at[pl.ds(base, TW)], rows_v)
    pltpu.sync_copy(p0_hbm.at[pl.ds(base, TW)], i0_v)
    pltpu.sync_copy(p1_hbm.at[pl.ds(base, TW)], i1_v)
    pltpu.sync_copy(p2_hbm.at[pl.ds(base, TW)], i2_v)
    d0 = pltpu.async_copy(rows_v, xs_hbm.at[i0_v], sem)
    d1 = pltpu.async_copy(rows_v, xs_hbm.at[i1_v], sem)
    d2 = pltpu.async_copy(rows_v, xs_hbm.at[i2_v], sem)
    d0.wait()
    d1.wait()
    d2.wait()


def _gath_kernel(ys_hbm, pf_hbm, yg_hbm, r0_v, r1_v, i0_v, i1_v, sem):
    wid = lax.axis_index("s") * 2 + lax.axis_index("c")
    CW = r0_v.shape[0]
    nchunk = pf_hbm.shape[0] // (NW * CW)
    rows = (r0_v, r1_v)
    idxs = (i0_v, i1_v)
    base0 = wid * (CW * nchunk)
    pltpu.sync_copy(pf_hbm.at[pl.ds(base0, CW)], i0_v)
    pend = pltpu.async_copy(ys_hbm.at[i0_v], r0_v, sem)
    for c in range(nchunk):
        if c + 1 < nchunk:
            nb = base0 + (c + 1) * CW
            pltpu.sync_copy(pf_hbm.at[pl.ds(nb, CW)], idxs[(c + 1) % 2])
            nxt = pltpu.async_copy(ys_hbm.at[idxs[(c + 1) % 2]],
                                   rows[(c + 1) % 2], sem)
        pend.wait()
        pltpu.sync_copy(rows[c % 2], yg_hbm.at[pl.ds(base0 + c * CW, CW)])
        if c + 1 < nchunk:
            pend = nxt


def kernel(x, Wg, bg, W1, b1, W2, b2, Ws1, bs1, Ws2, bs2, bias):
    o_shape = x.shape
    x2 = x.reshape(-1, D)
    T = x2.shape[0]
    PADN = NT * BT

    topw, pos, te, aux, xpk = pl.pallas_call(
        _gate_kernel,
        out_shape=[
            jax.ShapeDtypeStruct((T, K), jnp.float32),
            jax.ShapeDtypeStruct((T, K), jnp.int32),
            jax.ShapeDtypeStruct((1, NT), jnp.int32),
            jax.ShapeDtypeStruct((1, 1), jnp.float32),
            jax.ShapeDtypeStruct((T, D // 2), jnp.int32),
        ],
    )(x2, Wg, bg.reshape(1, E), bias.reshape(1, E))
    tile_expert = te.reshape(NT)

    TW = T // NW
    mesh = plsc.VectorSubcoreMesh(core_axis_name="c", subcore_axis_name="s")
    xs = pl.kernel(
        _disp_kernel,
        out_type=jax.ShapeDtypeStruct((PADN, D // 2), jnp.int32),
        mesh=mesh,
        scratch_types=[
            pltpu.VMEM((TW, D // 2), jnp.int32),
            pltpu.VMEM((TW,), jnp.int32),
            pltpu.VMEM((TW,), jnp.int32),
            pltpu.VMEM((TW,), jnp.int32),
            pltpu.SemaphoreType.DMA,
        ],
    )(xpk, pos[:, 0], pos[:, 1], pos[:, 2])

    grid_spec = pltpu.PrefetchScalarGridSpec(
        num_scalar_prefetch=1,
        grid=(NT,),
        in_specs=[
            pl.BlockSpec((BT, D // 2), lambda j, te: (j, 0)),
            pl.BlockSpec((1, H, D), lambda j, te: (te[j], 0, 0)),
            pl.BlockSpec((1, 1, H), lambda j, te: (te[j], 0, 0)),
            pl.BlockSpec((1, D, H), lambda j, te: (te[j], 0, 0)),
            pl.BlockSpec((1, 1, D), lambda j, te: (te[j], 0, 0)),
        ],
        out_specs=pl.BlockSpec((BT, D // 2), lambda j, te: (j, 0)),
    )
    ys = pl.pallas_call(
        _ffn_kernel,
        grid_spec=grid_spec,
        out_shape=jax.ShapeDtypeStruct((PADN, D // 2), jnp.int32),
    )(tile_expert, xs, W1, b1.reshape(E, 1, H), W2, b2.reshape(E, 1, D))

    CW = 96
    yg = pl.kernel(
        _gath_kernel,
        out_type=jax.ShapeDtypeStruct((T * K, D // 2), jnp.int32),
        mesh=mesh,
        scratch_types=[
            pltpu.VMEM((CW, D // 2), jnp.int32),
            pltpu.VMEM((CW, D // 2), jnp.int32),
            pltpu.VMEM((CW,), jnp.int32),
            pltpu.VMEM((CW,), jnp.int32),
            pltpu.SemaphoreType.DMA,
        ],
    )(ys, pos.reshape(T * K))

    BTC = 512
    out = pl.pallas_call(
        _combine_kernel,
        grid=(T // BTC,),
        in_specs=[
            pl.BlockSpec((BTC, D // 2), lambda i: (i, 0)),
            pl.BlockSpec((BTC, K * D // 2), lambda i: (i, 0)),
            pl.BlockSpec((BTC, K), lambda i: (i, 0)),
            pl.BlockSpec((H, D), lambda i: (0, 0)),
            pl.BlockSpec((1, H), lambda i: (0, 0)),
            pl.BlockSpec((D, H), lambda i: (0, 0)),
            pl.BlockSpec((1, D), lambda i: (0, 0)),
        ],
        out_specs=pl.BlockSpec((BTC, D), lambda i: (i, 0)),
        out_shape=jax.ShapeDtypeStruct((T, D), jnp.float32),
    )(xpk, yg.reshape(T, K * D // 2), topw, Ws1, bs1.reshape(1, H),
      Ws2, bs2.reshape(1, D))

    return out.reshape(o_shape), aux[0, 0]
